# baseline (speedup 1.0000x reference)
import jax
import jax.numpy as jnp
from jax.experimental import pallas as pl


_S_BLK = 256


def _add_kernel(in_ref, emb_ref, out_ref):
    out_ref[...] = in_ref[...] + emb_ref[...][None, :, :]


def kernel(inputs, embeddings):
    batch, seq_len, dim = inputs.shape
    pos = embeddings[:seq_len]
    grid = (seq_len // _S_BLK, batch)
    return pl.pallas_call(
        _add_kernel,
        grid=grid,
        in_specs=[
            pl.BlockSpec((1, _S_BLK, dim), lambda i, j: (j, i, 0)),
            pl.BlockSpec((_S_BLK, dim), lambda i, j: (i, 0)),
        ],
        out_specs=pl.BlockSpec((1, _S_BLK, dim), lambda i, j: (j, i, 0)),
        out_shape=jax.ShapeDtypeStruct((batch, seq_len, dim), inputs.dtype),
    )(inputs, pos)


# speedup vs baseline: 1.4759x; 1.4759x over previous
import jax
import jax.numpy as jnp
from jax.experimental import pallas as pl


_S_BLK = 512


def _add_kernel(in_ref, emb_ref, out_ref):
    out_ref[...] = in_ref[...] + emb_ref[...][None, :, :]


def kernel(inputs, embeddings):
    batch, seq_len, dim = inputs.shape
    pos = embeddings[:seq_len]
    grid = (seq_len // _S_BLK,)
    return pl.pallas_call(
        _add_kernel,
        grid=grid,
        in_specs=[
            pl.BlockSpec((batch, _S_BLK, dim), lambda i: (0, i, 0)),
            pl.BlockSpec((_S_BLK, dim), lambda i: (i, 0)),
        ],
        out_specs=pl.BlockSpec((batch, _S_BLK, dim), lambda i: (0, i, 0)),
        out_shape=jax.ShapeDtypeStruct((batch, seq_len, dim), inputs.dtype),
    )(inputs, pos)
